# Initial kernel scaffold; baseline (speedup 1.0000x reference)
import jax
import jax.numpy as jnp
from jax.experimental import pallas as pl


def kernel(x, pos_table):
    raise NotImplementedError("write your pallas kernel here")


# speedup vs baseline: 1.7231x; 1.7231x over previous
import functools

import jax
import jax.numpy as jnp
from jax.experimental import pallas as pl

BATCH = 4
SEQ_LEN = 8192
D_MODEL = 1024
SB = 512


def _add_kernel(x_ref, pos_ref, out_ref):
    out_ref[...] = x_ref[...] + pos_ref[...][None, :, :]


@jax.jit
def kernel(x, pos_table):
    grid = (SEQ_LEN // SB,)
    return pl.pallas_call(
        _add_kernel,
        grid=grid,
        in_specs=[
            pl.BlockSpec((BATCH, SB, D_MODEL), lambda i: (0, i, 0)),
            pl.BlockSpec((SB, D_MODEL), lambda i: (i, 0)),
        ],
        out_specs=pl.BlockSpec((BATCH, SB, D_MODEL), lambda i: (0, i, 0)),
        out_shape=jax.ShapeDtypeStruct((BATCH, SEQ_LEN, D_MODEL), x.dtype),
    )(x, pos_table)
